# baseline (speedup 1.0000x reference)
import functools

import jax
import jax.numpy as jnp
from jax import lax
from jax.experimental import pallas as pl
from jax.experimental.pallas import tpu as pltpu
from jax.experimental.pallas import tpu_sc as plsc

NUM_CORES = 2
NUM_SUBCORES = 16
LANES = 16
NW = NUM_CORES * NUM_SUBCORES
ROWS_PER_SUPER = 8
SUPER = 128
CHUNK = 128
NBUF = 2


def _mf_body(users_hbm, items_hbm, utab_hbm, itab_hbm, out_hbm,
             idx_u, idx_i, sidx_u, sidx_i, ubuf, ibuf, outv, sems):
    b_per_w = idx_u.shape[0]
    wid = lax.axis_index("s") * NUM_CORES + lax.axis_index("c")
    base = wid * b_per_w

    pltpu.sync_copy(users_hbm.at[pl.ds(base, b_per_w)], idx_u)
    pltpu.sync_copy(items_hbm.at[pl.ds(base, b_per_w)], idx_i)

    @pl.loop(0, b_per_w // LANES)
    def _sidx(j):
        sl = pl.ds(j * LANES, LANES)
        sidx_u[sl] = lax.shift_right_logical(idx_u[sl], 3)
        sidx_i[sl] = lax.shift_right_logical(idx_i[sl], 3)

    nchunk = b_per_w // CHUNK

    def start(c):
        buf = c % NBUF
        sl = pl.ds(c * CHUNK, CHUNK)
        cp_u = pltpu.async_copy(utab_hbm.at[sidx_u.at[sl]], ubuf.at[buf],
                                sems.at[buf, 0])
        cp_i = pltpu.async_copy(itab_hbm.at[sidx_i.at[sl]], ibuf.at[buf],
                                sems.at[buf, 1])
        return cp_u, cp_i

    def compute(c):
        buf = c % NBUF
        for b in range(CHUNK // LANES):
            goff = c * CHUNK + b * LANES
            rows = b * LANES + lax.iota(jnp.int32, LANES)
            ucols = (idx_u[pl.ds(goff, LANES)] & 7) * LANES
            icols = (idx_i[pl.ds(goff, LANES)] & 7) * LANES
            acc = jnp.zeros((LANES,), jnp.float32)
            for d in range(LANES):
                uu = plsc.load_gather(ubuf.at[buf], [rows, ucols + d])
                vv = plsc.load_gather(ibuf.at[buf], [rows, icols + d])
                acc = acc + uu * vv
            outv[pl.ds(goff, LANES)] = 1.0 / (1.0 + jnp.exp(-acc))

    pending = {0: start(0)}
    for c in range(nchunk):
        if c + 1 < nchunk and (c + 1) % NBUF != c % NBUF:
            pending[c + 1] = start(c + 1)
        cp_u, cp_i = pending.pop(c)
        cp_u.wait()
        cp_i.wait()
        compute(c)

    pltpu.sync_copy(outv, out_hbm.at[pl.ds(base, b_per_w)])


def kernel(users, items, user_table, item_table):
    batch = users.shape[0]
    b_per_w = batch // NW
    utab = user_table.reshape(-1, SUPER)
    itab = item_table.reshape(-1, SUPER)
    mesh = plsc.VectorSubcoreMesh(
        core_axis_name="c", subcore_axis_name="s",
        num_cores=NUM_CORES, num_subcores=NUM_SUBCORES)
    run = functools.partial(
        pl.kernel,
        out_type=jax.ShapeDtypeStruct((batch,), jnp.float32),
        mesh=mesh,
        compiler_params=pltpu.CompilerParams(needs_layout_passes=False),
        scratch_types=[
            pltpu.VMEM((b_per_w,), jnp.int32),
            pltpu.VMEM((b_per_w,), jnp.int32),
            pltpu.VMEM((b_per_w,), jnp.int32),
            pltpu.VMEM((b_per_w,), jnp.int32),
            pltpu.VMEM((NBUF, CHUNK, SUPER), jnp.float32),
            pltpu.VMEM((NBUF, CHUNK, SUPER), jnp.float32),
            pltpu.VMEM((b_per_w,), jnp.float32),
            pltpu.SemaphoreType.DMA((NBUF, 2)),
        ],
    )(_mf_body)
    return run(users, items, utab, itab)


# speedup vs baseline: 5.9153x; 5.9153x over previous
import functools

import jax
import jax.numpy as jnp
from jax import lax
from jax.experimental import pallas as pl
from jax.experimental.pallas import tpu as pltpu
from jax.experimental.pallas import tpu_sc as plsc

NUM_CORES = 2
NUM_SUBCORES = 16
LANES = 16
NW = NUM_CORES * NUM_SUBCORES
DIM = 16
GROUP = 16
BLK = 128


def _fetch(tabT_hbm, dst, r, sem):
    rb = lax.shift_right_logical(r, 7)
    off = pl.multiple_of(rb * BLK, BLK)
    return pltpu.async_copy(tabT_hbm.at[:, pl.ds(off, BLK)], dst, sem)


def _mf_body(users_hbm, items_hbm, utabT_hbm, itabT_hbm, out_hbm,
             idx_u, idx_i, ublk, iblk, outv, sem_u, sem_i):
    b_per_w = idx_u.shape[0]
    wid = lax.axis_index("s") * NUM_CORES + lax.axis_index("c")
    base = wid * b_per_w

    pltpu.sync_copy(users_hbm.at[pl.ds(base, b_per_w)], idx_u)
    pltpu.sync_copy(items_hbm.at[pl.ds(base, b_per_w)], idx_i)

    jvec = lax.iota(jnp.int32, LANES) & (GROUP - 1)
    dvecs = [jnp.full((LANES,), d, jnp.int32) for d in range(DIM)]

    @pl.loop(0, b_per_w // GROUP)
    def _groups(p):
        sl = pl.ds(p * GROUP, GROUP)
        vec_u = idx_u[sl]
        vec_i = idx_i[sl]
        cps = []
        for k in range(GROUP):
            cps.append(_fetch(utabT_hbm, ublk.at[k], vec_u[k], sem_u))
            cps.append(_fetch(itabT_hbm, iblk.at[k], vec_i[k], sem_i))
        for cp in cps:
            cp.wait()

        lane_u = vec_u & (BLK - 1)
        lane_i = vec_i & (BLK - 1)
        acc = jnp.zeros((LANES,), jnp.float32)
        for d in range(DIM):
            uu = plsc.load_gather(ublk, [jvec, dvecs[d], lane_u])
            vv = plsc.load_gather(iblk, [jvec, dvecs[d], lane_i])
            acc = acc + uu * vv
        outv[sl] = 1.0 / (1.0 + jnp.exp(-acc))

    pltpu.sync_copy(outv, out_hbm.at[pl.ds(base, b_per_w)])


def kernel(users, items, user_table, item_table):
    batch = users.shape[0]
    b_per_w = batch // NW
    utabT = user_table.T
    itabT = item_table.T
    mesh = plsc.VectorSubcoreMesh(
        core_axis_name="c", subcore_axis_name="s",
        num_cores=NUM_CORES, num_subcores=NUM_SUBCORES)
    run = functools.partial(
        pl.kernel,
        out_type=jax.ShapeDtypeStruct((batch,), jnp.float32),
        mesh=mesh,
        compiler_params=pltpu.CompilerParams(needs_layout_passes=False),
        scratch_types=[
            pltpu.VMEM((b_per_w,), jnp.int32),
            pltpu.VMEM((b_per_w,), jnp.int32),
            pltpu.VMEM((GROUP, DIM, BLK), jnp.float32),
            pltpu.VMEM((GROUP, DIM, BLK), jnp.float32),
            pltpu.VMEM((b_per_w,), jnp.float32),
            pltpu.SemaphoreType.DMA,
            pltpu.SemaphoreType.DMA,
        ],
    )(_mf_body)
    return run(users, items, utabT, itabT)


# speedup vs baseline: 6.4098x; 1.0836x over previous
import functools

import jax
import jax.numpy as jnp
from jax import lax
from jax.experimental import pallas as pl
from jax.experimental.pallas import tpu as pltpu
from jax.experimental.pallas import tpu_sc as plsc

NUM_CORES = 2
NUM_SUBCORES = 16
LANES = 16
NW = NUM_CORES * NUM_SUBCORES
DIM = 16
GROUP = 8
NBUF = 2
BLK = 128


def _block_src(tabT_hbm, r):
    rb = lax.shift_right_logical(r, 7)
    off = pl.multiple_of(rb * BLK, BLK)
    return tabT_hbm.at[:, pl.ds(off, BLK)]


def _mf_body(users_hbm, items_hbm, utabT_hbm, itabT_hbm, out_hbm,
             idx_u, idx_i, ublk, iblk, outv, sem_u, sem_i):
    b_per_w = idx_u.shape[0]
    wid = lax.axis_index("s") * NUM_CORES + lax.axis_index("c")
    base = wid * b_per_w

    pltpu.sync_copy(users_hbm.at[pl.ds(base, b_per_w)], idx_u)
    pltpu.sync_copy(items_hbm.at[pl.ds(base, b_per_w)], idx_i)

    jvec = lax.iota(jnp.int32, LANES) & (GROUP - 1)
    m_lo = lax.iota(jnp.int32, LANES) < GROUP
    dvecs = [jnp.full((LANES,), d, jnp.int32) for d in range(DIM)]

    def fire(vec_u, vec_i, half, buf):
        for j in range(GROUP):
            k = half * GROUP + j
            pltpu.async_copy(_block_src(utabT_hbm, vec_u[k]),
                             ublk.at[buf, j], sem_u.at[buf])
            pltpu.async_copy(_block_src(itabT_hbm, vec_i[k]),
                             iblk.at[buf, j], sem_i.at[buf])

    def drain(vec_u, vec_i, half, buf):
        for j in range(GROUP):
            k = half * GROUP + j
            pltpu.make_async_copy(_block_src(utabT_hbm, vec_u[k]),
                                  ublk.at[buf, j], sem_u.at[buf]).wait()
            pltpu.make_async_copy(_block_src(itabT_hbm, vec_i[k]),
                                  iblk.at[buf, j], sem_i.at[buf]).wait()

    def extract(vec_u, vec_i, buf):
        lane_u = vec_u & (BLK - 1)
        lane_i = vec_i & (BLK - 1)
        acc = jnp.zeros((LANES,), jnp.float32)
        for d in range(DIM):
            uu = plsc.load_gather(ublk.at[buf], [jvec, dvecs[d], lane_u])
            vv = plsc.load_gather(iblk.at[buf], [jvec, dvecs[d], lane_i])
            acc = acc + uu * vv
        return acc

    fire(idx_u[pl.ds(0, LANES)], idx_i[pl.ds(0, LANES)], 0, 0)

    npair = b_per_w // LANES

    @pl.loop(0, npair)
    def _pairs(p):
        sl = pl.ds(p * LANES, LANES)
        vec_u = idx_u[sl]
        vec_i = idx_i[sl]
        fire(vec_u, vec_i, 1, 1)
        drain(vec_u, vec_i, 0, 0)
        acc_lo = extract(vec_u, vec_i, 0)

        @pl.when(p < npair - 1)
        def _next():
            nsl = pl.ds((p + 1) * LANES, LANES)
            fire(idx_u[nsl], idx_i[nsl], 0, 0)

        drain(vec_u, vec_i, 1, 1)
        acc_hi = extract(vec_u, vec_i, 1)
        acc = jnp.where(m_lo, acc_lo, acc_hi)
        outv[sl] = 1.0 / (1.0 + jnp.exp(-acc))

    pltpu.sync_copy(outv, out_hbm.at[pl.ds(base, b_per_w)])


def kernel(users, items, user_table, item_table):
    batch = users.shape[0]
    b_per_w = batch // NW
    utabT = user_table.T
    itabT = item_table.T
    mesh = plsc.VectorSubcoreMesh(
        core_axis_name="c", subcore_axis_name="s",
        num_cores=NUM_CORES, num_subcores=NUM_SUBCORES)
    run = functools.partial(
        pl.kernel,
        out_type=jax.ShapeDtypeStruct((batch,), jnp.float32),
        mesh=mesh,
        compiler_params=pltpu.CompilerParams(needs_layout_passes=False),
        scratch_types=[
            pltpu.VMEM((b_per_w,), jnp.int32),
            pltpu.VMEM((b_per_w,), jnp.int32),
            pltpu.VMEM((NBUF, GROUP, DIM, BLK), jnp.float32),
            pltpu.VMEM((NBUF, GROUP, DIM, BLK), jnp.float32),
            pltpu.VMEM((b_per_w,), jnp.float32),
            pltpu.SemaphoreType.DMA((NBUF,)),
            pltpu.SemaphoreType.DMA((NBUF,)),
        ],
    )(_mf_body)
    return run(users, items, utabT, itabT)


# speedup vs baseline: 6.4178x; 1.0012x over previous
import functools

import jax
import jax.numpy as jnp
from jax import lax
from jax.experimental import pallas as pl
from jax.experimental.pallas import tpu as pltpu
from jax.experimental.pallas import tpu_sc as plsc

NUM_CORES = 2
NUM_SUBCORES = 16
LANES = 16
NW = NUM_CORES * NUM_SUBCORES
DIM = 16
GROUP = 8
NBUF = 2
BLK = 128


def _mf_body(users_hbm, items_hbm, utabT_hbm, itabT_hbm, out_hbm,
             idx_u, idx_i, ublk, iblk, outv, sem_u, sem_i):
    b_per_w = idx_u.shape[0]
    wid = lax.axis_index("s") * NUM_CORES + lax.axis_index("c")
    base = wid * b_per_w

    pltpu.sync_copy(users_hbm.at[pl.ds(base, b_per_w)], idx_u)
    pltpu.sync_copy(items_hbm.at[pl.ds(base, b_per_w)], idx_i)

    jslot = (lax.iota(jnp.int32, LANES) & (GROUP - 1)) * BLK
    m_lo = lax.iota(jnp.int32, LANES) < GROUP
    dvecs = [jnp.full((LANES,), d, jnp.int32) for d in range(DIM)]

    def fire(vec_u, vec_i, half, buf):
        off_u = lax.shift_right_logical(vec_u, 7) * BLK
        off_i = lax.shift_right_logical(vec_i, 7) * BLK
        for j in range(GROUP):
            k = half * GROUP + j
            ou = pl.multiple_of(off_u[k], BLK)
            oi = pl.multiple_of(off_i[k], BLK)
            pltpu.async_copy(utabT_hbm.at[:, pl.ds(ou, BLK)],
                             ublk.at[buf].at[:, pl.ds(j * BLK, BLK)],
                             sem_u.at[buf])
            pltpu.async_copy(itabT_hbm.at[:, pl.ds(oi, BLK)],
                             iblk.at[buf].at[:, pl.ds(j * BLK, BLK)],
                             sem_i.at[buf])

    def drain(buf):
        pltpu.make_async_copy(utabT_hbm.at[:, pl.ds(0, GROUP * BLK)],
                              ublk.at[buf], sem_u.at[buf]).wait()
        pltpu.make_async_copy(itabT_hbm.at[:, pl.ds(0, GROUP * BLK)],
                              iblk.at[buf], sem_i.at[buf]).wait()

    def extract(vec_u, vec_i, buf):
        lane_u = jslot + (vec_u & (BLK - 1))
        lane_i = jslot + (vec_i & (BLK - 1))
        acc = jnp.zeros((LANES,), jnp.float32)
        for d in range(DIM):
            uu = plsc.load_gather(ublk.at[buf], [dvecs[d], lane_u])
            vv = plsc.load_gather(iblk.at[buf], [dvecs[d], lane_i])
            acc = acc + uu * vv
        return acc

    fire(idx_u[pl.ds(0, LANES)], idx_i[pl.ds(0, LANES)], 0, 0)

    npair = b_per_w // LANES

    @pl.loop(0, npair)
    def _pairs(p):
        sl = pl.ds(p * LANES, LANES)
        vec_u = idx_u[sl]
        vec_i = idx_i[sl]
        fire(vec_u, vec_i, 1, 1)
        drain(0)
        acc_lo = extract(vec_u, vec_i, 0)

        @pl.when(p < npair - 1)
        def _next():
            nsl = pl.ds((p + 1) * LANES, LANES)
            fire(idx_u[nsl], idx_i[nsl], 0, 0)

        drain(1)
        acc_hi = extract(vec_u, vec_i, 1)
        acc = jnp.where(m_lo, acc_lo, acc_hi)
        outv[sl] = 1.0 / (1.0 + jnp.exp(-acc))

    pltpu.sync_copy(outv, out_hbm.at[pl.ds(base, b_per_w)])


def kernel(users, items, user_table, item_table):
    batch = users.shape[0]
    b_per_w = batch // NW
    utabT = user_table.T
    itabT = item_table.T
    mesh = plsc.VectorSubcoreMesh(
        core_axis_name="c", subcore_axis_name="s",
        num_cores=NUM_CORES, num_subcores=NUM_SUBCORES)
    run = functools.partial(
        pl.kernel,
        out_type=jax.ShapeDtypeStruct((batch,), jnp.float32),
        mesh=mesh,
        compiler_params=pltpu.CompilerParams(needs_layout_passes=False),
        scratch_types=[
            pltpu.VMEM((b_per_w,), jnp.int32),
            pltpu.VMEM((b_per_w,), jnp.int32),
            pltpu.VMEM((NBUF, DIM, GROUP * BLK), jnp.float32),
            pltpu.VMEM((NBUF, DIM, GROUP * BLK), jnp.float32),
            pltpu.VMEM((b_per_w,), jnp.float32),
            pltpu.SemaphoreType.DMA((NBUF,)),
            pltpu.SemaphoreType.DMA((NBUF,)),
        ],
    )(_mf_body)
    return run(users, items, utabT, itabT)
